# initial kernel scaffold (unmeasured)
import jax
import jax.numpy as jnp
from jax import lax
from jax.experimental import pallas as pl
from jax.experimental.pallas import tpu as pltpu

N_DEV = 4
SQ = 1024
SKV_LOC = 1024
H_LOC = 8
H_TOT = 32
DH = 128
DM = 1024
NB = 4
BLK = 64
GQ = SQ // NB
GKV = SKV_LOC // NB
QROWS = SQ // N_DEV
SCALE = 0.08838834764831843

_MESH = pl.DeviceIdType.MESH if hasattr(pl, "DeviceIdType") else pltpu.DeviceIdType.MESH
_sem_signal = pl.semaphore_signal if hasattr(pl, "semaphore_signal") else pltpu.semaphore_signal
_sem_wait = pl.semaphore_wait if hasattr(pl, "semaphore_wait") else pltpu.semaphore_wait
_Params = getattr(pltpu, "CompilerParams", None) or getattr(pltpu, "TPUCompilerParams")
_ANY = getattr(pltpu, "ANY", None)
if _ANY is None:
    _ANY = pltpu.MemorySpace.ANY if hasattr(pltpu, "MemorySpace") else pltpu.TPUMemorySpace.ANY


def kernel(x, Wq, K_ext, V_ext, Wo):
    def body(x_hbm, wq_hbm, k_hbm, v_hbm, wo_hbm, out_ref,
             kv_send, k_full, v_full, stage, wstage,
             x_bf, wq_bf, wo_bf, q_bf, ctx_buf, p_send,
             rs_recv, ag_own, ag_recv,
             stage_sems, wstage_sems, own_sems,
             k_send_sems, k_recv_sems, v_send_sems, v_recv_sems,
             rs_send_sems, rs_recv_sems, ag_send_sems, ag_recv_sems):
        my = lax.axis_index("i")

        bar = pltpu.get_barrier_semaphore()
        for off in (1, 2, 3):
            _sem_signal(bar, inc=1,
                        device_id=(lax.rem(my + off, N_DEV),),
                        device_id_type=_MESH)
        _sem_wait(bar, N_DEV - 1)

        items = [(t, b) for t in (0, 1) for b in range(SKV_LOC // BLK)]

        def _kv_src(t, b):
            ref = k_hbm if t == 0 else v_hbm
            return ref.at[0, pl.ds(b * BLK, BLK), :, :]

        cps = [None, None]
        cps[0] = pltpu.make_async_copy(_kv_src(*items[0]), stage.at[0],
                                       stage_sems.at[0])
        cps[0].start()
        for i, (t, b) in enumerate(items):
            slot = i % 2
            if i + 1 < len(items):
                ns = (i + 1) % 2
                cps[ns] = pltpu.make_async_copy(_kv_src(*items[i + 1]),
                                                stage.at[ns], stage_sems.at[ns])
                cps[ns].start()
            cps[slot].wait()
            r, n = b % NB, b // NB
            kv_send[t, r, n * BLK:(n + 1) * BLK, :, :] = (
                stage[slot].astype(jnp.bfloat16))

        k_own = pltpu.make_async_copy(
            kv_send.at[0, :, :, pl.ds(my * H_LOC, H_LOC), :],
            k_full.at[:, 3 * GKV:4 * GKV, :, :], own_sems.at[0])
        v_own = pltpu.make_async_copy(
            kv_send.at[1, :, :, pl.ds(my * H_LOC, H_LOC), :],
            v_full.at[:, 3 * GKV:4 * GKV, :, :], own_sems.at[1])
        k_own.start()
        v_own.start()

        kv_rdmas = []
        for off in (1, 2, 3):
            p = lax.rem(my + off, N_DEV)
            lo, hi = (off - 1) * GKV, off * GKV
            for t, full, ssem, rsem in (
                    (0, k_full, k_send_sems, k_recv_sems),
                    (1, v_full, v_send_sems, v_recv_sems)):
                rd = pltpu.make_async_remote_copy(
                    src_ref=kv_send.at[t, :, :, pl.ds(p * H_LOC, H_LOC), :],
                    dst_ref=full.at[:, lo:hi, :, :],
                    send_sem=ssem.at[off - 1],
                    recv_sem=rsem.at[off - 1],
                    device_id=(p,),
                    device_id_type=_MESH)
                rd.start()
                kv_rdmas.append(rd)

        mats = [(x_hbm.at[0], x_bf), (wq_hbm.at[:, :], wq_bf),
                (wo_hbm.at[:, :], wo_bf)]
        witems = [(mi, c) for mi in range(3) for c in range(4)]
        wps = [None, None]
        wps[0] = pltpu.make_async_copy(
            mats[0][0].at[pl.ds(0, 256), :], wstage.at[0], wstage_sems.at[0])
        wps[0].start()
        for i, (mi, c) in enumerate(witems):
            slot = i % 2
            if i + 1 < len(witems):
                nmi, nc = witems[i + 1]
                ns = (i + 1) % 2
                wps[ns] = pltpu.make_async_copy(
                    mats[nmi][0].at[pl.ds(nc * 256, 256), :],
                    wstage.at[ns], wstage_sems.at[ns])
                wps[ns].start()
            wps[slot].wait()
            mats[mi][1][c * 256:(c + 1) * 256, :] = (
                wstage[slot].astype(jnp.bfloat16))

        for c in range(4):
            qc = jnp.dot(x_bf[c * 256:(c + 1) * 256, :], wq_bf[:, :],
                         preferred_element_type=jnp.float32)
            q_bf[c * 256:(c + 1) * 256, :] = qc.astype(jnp.bfloat16)

        k_own.wait()
        v_own.wait()
        for rd in kv_rdmas:
            rd.wait_recv()

        for r in range(NB):
            for h in range(H_LOC):
                q_rh = jnp.concatenate(
                    [q_bf[(NB * m + r) * BLK:(NB * m + r + 1) * BLK,
                          h * DH:(h + 1) * DH] for m in range(4)], axis=0)
                k_rh = k_full[r, :, h, :]
                s = lax.dot_general(
                    q_rh, k_rh, (((1,), (1,)), ((), ())),
                    preferred_element_type=jnp.float32)
                s = s * SCALE
                mx = jnp.max(s, axis=1, keepdims=True)
                w = jnp.exp(s - mx)
                den = jnp.sum(w, axis=1, keepdims=True)
                pw = (w / den).astype(jnp.bfloat16)
                v_rh = v_full[r, :, h, :]
                ctx = jnp.dot(pw, v_rh,
                              preferred_element_type=jnp.float32)
                ctxb = ctx.astype(jnp.bfloat16)
                for m in range(4):
                    ctx_buf[(NB * m + r) * BLK:(NB * m + r + 1) * BLK,
                            h * DH:(h + 1) * DH] = ctxb[m * BLK:(m + 1) * BLK]

        for c in range(4):
            pc = jnp.dot(ctx_buf[c * 256:(c + 1) * 256, :], wo_bf[:, :],
                         preferred_element_type=jnp.float32)
            p_send[c * 256:(c + 1) * 256, :] = pc.astype(jnp.bfloat16)

        rs_rdmas = []
        for off in (1, 2, 3):
            p = lax.rem(my + off, N_DEV)
            rd = pltpu.make_async_remote_copy(
                src_ref=p_send.at[pl.ds(p * QROWS, QROWS), :],
                dst_ref=rs_recv.at[off - 1],
                send_sem=rs_send_sems.at[off - 1],
                recv_sem=rs_recv_sems.at[off - 1],
                device_id=(p,),
                device_id_type=_MESH)
            rd.start()
            rs_rdmas.append(rd)
        for rd in rs_rdmas:
            rd.wait_recv()
        acc = p_send[pl.ds(my * QROWS, QROWS), :].astype(jnp.float32)
        for o in range(3):
            acc = acc + rs_recv[o].astype(jnp.float32)
        ag_own[:, :] = acc.astype(jnp.bfloat16)

        ag_rdmas = []
        for off in (1, 2, 3):
            p = lax.rem(my + off, N_DEV)
            rd = pltpu.make_async_remote_copy(
                src_ref=ag_own.at[:, :],
                dst_ref=ag_recv.at[off - 1],
                send_sem=ag_send_sems.at[off - 1],
                recv_sem=ag_recv_sems.at[off - 1],
                device_id=(p,),
                device_id_type=_MESH)
            rd.start()
            ag_rdmas.append(rd)
        out_ref[0, pl.ds(my * QROWS, QROWS), :] = acc
        for rd in ag_rdmas:
            rd.wait_recv()
        for off in (1, 2, 3):
            j = lax.rem(my + (N_DEV - off), N_DEV)
            out_ref[0, pl.ds(j * QROWS, QROWS), :] = (
                ag_recv[off - 1].astype(jnp.float32))

        for rd in kv_rdmas + rs_rdmas + ag_rdmas:
            rd.wait_send()

    bf = jnp.bfloat16
    f32 = jnp.float32
    return pl.pallas_call(
        body,
        out_shape=jax.ShapeDtypeStruct((1, SQ, DM), f32),
        in_specs=[pl.BlockSpec(memory_space=_ANY)] * 5,
        out_specs=pl.BlockSpec(memory_space=pltpu.VMEM),
        scratch_shapes=[
            pltpu.VMEM((2, NB, GKV, H_TOT, DH), bf),
            pltpu.VMEM((NB, N_DEV * GKV, H_LOC, DH), bf),
            pltpu.VMEM((NB, N_DEV * GKV, H_LOC, DH), bf),
            pltpu.VMEM((2, BLK, H_TOT, DH), f32),
            pltpu.VMEM((2, 256, DM), f32),
            pltpu.VMEM((DM, DM), bf),
            pltpu.VMEM((DM, DM), bf),
            pltpu.VMEM((DM, DM), bf),
            pltpu.VMEM((SQ, H_LOC * DH), bf),
            pltpu.VMEM((SQ, H_LOC * DH), bf),
            pltpu.VMEM((SQ, DM), bf),
            pltpu.VMEM((3, QROWS, DM), bf),
            pltpu.VMEM((QROWS, DM), bf),
            pltpu.VMEM((3, QROWS, DM), bf),
            pltpu.SemaphoreType.DMA((2,)),
            pltpu.SemaphoreType.DMA((2,)),
            pltpu.SemaphoreType.DMA((2,)),
            pltpu.SemaphoreType.DMA((3,)),
            pltpu.SemaphoreType.DMA((3,)),
            pltpu.SemaphoreType.DMA((3,)),
            pltpu.SemaphoreType.DMA((3,)),
            pltpu.SemaphoreType.DMA((3,)),
            pltpu.SemaphoreType.DMA((3,)),
            pltpu.SemaphoreType.DMA((3,)),
            pltpu.SemaphoreType.DMA((3,)),
        ],
        compiler_params=_Params(collective_id=0),
    )(x, Wq, K_ext, V_ext, Wo)


# baseline (device time: 201434 ns/iter reference)
import jax
import jax.numpy as jnp
from jax import lax
from jax.experimental import pallas as pl
from jax.experimental.pallas import tpu as pltpu

N_DEV = 4
SQ = 1024
SKV_LOC = 1024
H_LOC = 8
H_TOT = 32
DH = 128
DM = 1024
NB = 4
BLK = 64
GQ = SQ // NB
GKV = SKV_LOC // NB
QROWS = SQ // N_DEV
SCALE = 0.08838834764831843

_MESH = pl.DeviceIdType.MESH if hasattr(pl, "DeviceIdType") else pltpu.DeviceIdType.MESH
_sem_signal = pl.semaphore_signal if hasattr(pl, "semaphore_signal") else pltpu.semaphore_signal
_sem_wait = pl.semaphore_wait if hasattr(pl, "semaphore_wait") else pltpu.semaphore_wait
_Params = getattr(pltpu, "CompilerParams", None) or getattr(pltpu, "TPUCompilerParams")
_ANY = pl.ANY


def kernel(x, Wq, K_ext, V_ext, Wo):
    def body(x_hbm, wq_hbm, k_hbm, v_hbm, wo_hbm, out_ref,
             kv_send, k_full, v_full, stage, wstage,
             x_bf, wq_bf, wo_bf, q_bf, ctx_buf, p_send,
             rs_recv, ag_own, ag_recv,
             stage_sems, wstage_sems, own_sems,
             k_send_sems, k_recv_sems, v_send_sems, v_recv_sems,
             rs_send_sems, rs_recv_sems, ag_send_sems, ag_recv_sems):
        my = lax.axis_index("i")

        bar = pltpu.get_barrier_semaphore()
        for off in (1, 2, 3):
            _sem_signal(bar, inc=1,
                        device_id=(lax.rem(my + off, N_DEV),),
                        device_id_type=_MESH)
        _sem_wait(bar, N_DEV - 1)

        items = [(t, b) for t in (0, 1) for b in range(SKV_LOC // BLK)]

        def _kv_src(t, b):
            ref = k_hbm if t == 0 else v_hbm
            return ref.at[0, pl.ds(b * BLK, BLK), :, :]

        cps = [None, None]
        cps[0] = pltpu.make_async_copy(_kv_src(*items[0]), stage.at[0],
                                       stage_sems.at[0])
        cps[0].start()
        for i, (t, b) in enumerate(items):
            slot = i % 2
            if i + 1 < len(items):
                ns = (i + 1) % 2
                cps[ns] = pltpu.make_async_copy(_kv_src(*items[i + 1]),
                                                stage.at[ns], stage_sems.at[ns])
                cps[ns].start()
            cps[slot].wait()
            r, n = b % NB, b // NB
            kv_send[t, r, n * BLK:(n + 1) * BLK, :, :] = (
                stage[slot].astype(jnp.bfloat16))

        k_own = pltpu.make_async_copy(
            kv_send.at[0, :, :, pl.ds(my * H_LOC, H_LOC), :],
            k_full.at[:, 3 * GKV:4 * GKV, :, :], own_sems.at[0])
        v_own = pltpu.make_async_copy(
            kv_send.at[1, :, :, pl.ds(my * H_LOC, H_LOC), :],
            v_full.at[:, 3 * GKV:4 * GKV, :, :], own_sems.at[1])
        k_own.start()
        v_own.start()

        kv_rdmas = []
        for off in (1, 2, 3):
            p = lax.rem(my + off, N_DEV)
            lo, hi = (off - 1) * GKV, off * GKV
            for t, full, ssem, rsem in (
                    (0, k_full, k_send_sems, k_recv_sems),
                    (1, v_full, v_send_sems, v_recv_sems)):
                rd = pltpu.make_async_remote_copy(
                    src_ref=kv_send.at[t, :, :, pl.ds(p * H_LOC, H_LOC), :],
                    dst_ref=full.at[:, lo:hi, :, :],
                    send_sem=ssem.at[off - 1],
                    recv_sem=rsem.at[off - 1],
                    device_id=(p,),
                    device_id_type=_MESH)
                rd.start()
                kv_rdmas.append(rd)

        mats = [(x_hbm.at[0], x_bf), (wq_hbm.at[:, :], wq_bf),
                (wo_hbm.at[:, :], wo_bf)]
        witems = [(mi, c) for mi in range(3) for c in range(4)]
        wps = [None, None]
        wps[0] = pltpu.make_async_copy(
            mats[0][0].at[pl.ds(0, 256), :], wstage.at[0], wstage_sems.at[0])
        wps[0].start()
        for i, (mi, c) in enumerate(witems):
            slot = i % 2
            if i + 1 < len(witems):
                nmi, nc = witems[i + 1]
                ns = (i + 1) % 2
                wps[ns] = pltpu.make_async_copy(
                    mats[nmi][0].at[pl.ds(nc * 256, 256), :],
                    wstage.at[ns], wstage_sems.at[ns])
                wps[ns].start()
            wps[slot].wait()
            mats[mi][1][c * 256:(c + 1) * 256, :] = (
                wstage[slot].astype(jnp.bfloat16))

        for c in range(4):
            qc = jnp.dot(x_bf[c * 256:(c + 1) * 256, :], wq_bf[:, :],
                         preferred_element_type=jnp.float32)
            q_bf[c * 256:(c + 1) * 256, :] = qc.astype(jnp.bfloat16)

        k_own.wait()
        v_own.wait()
        for rd in kv_rdmas:
            rd.wait_recv()

        for r in range(NB):
            for h in range(H_LOC):
                q_rh = jnp.concatenate(
                    [q_bf[(NB * m + r) * BLK:(NB * m + r + 1) * BLK,
                          h * DH:(h + 1) * DH] for m in range(4)], axis=0)
                k_rh = k_full[r, :, h, :]
                s = lax.dot_general(
                    q_rh, k_rh, (((1,), (1,)), ((), ())),
                    preferred_element_type=jnp.float32)
                s = s * SCALE
                mx = jnp.max(s, axis=1, keepdims=True)
                w = jnp.exp(s - mx)
                den = jnp.sum(w, axis=1, keepdims=True)
                pw = (w / den).astype(jnp.bfloat16)
                v_rh = v_full[r, :, h, :]
                ctx = jnp.dot(pw, v_rh,
                              preferred_element_type=jnp.float32)
                ctxb = ctx.astype(jnp.bfloat16)
                for m in range(4):
                    ctx_buf[(NB * m + r) * BLK:(NB * m + r + 1) * BLK,
                            h * DH:(h + 1) * DH] = ctxb[m * BLK:(m + 1) * BLK]

        for c in range(4):
            pc = jnp.dot(ctx_buf[c * 256:(c + 1) * 256, :], wo_bf[:, :],
                         preferred_element_type=jnp.float32)
            p_send[c * 256:(c + 1) * 256, :] = pc.astype(jnp.bfloat16)

        rs_rdmas = []
        for off in (1, 2, 3):
            p = lax.rem(my + off, N_DEV)
            rd = pltpu.make_async_remote_copy(
                src_ref=p_send.at[pl.ds(p * QROWS, QROWS), :],
                dst_ref=rs_recv.at[off - 1],
                send_sem=rs_send_sems.at[off - 1],
                recv_sem=rs_recv_sems.at[off - 1],
                device_id=(p,),
                device_id_type=_MESH)
            rd.start()
            rs_rdmas.append(rd)
        for rd in rs_rdmas:
            rd.wait_recv()
        acc = p_send[pl.ds(my * QROWS, QROWS), :].astype(jnp.float32)
        for o in range(3):
            acc = acc + rs_recv[o].astype(jnp.float32)
        ag_own[:, :] = acc.astype(jnp.bfloat16)

        ag_rdmas = []
        for off in (1, 2, 3):
            p = lax.rem(my + off, N_DEV)
            rd = pltpu.make_async_remote_copy(
                src_ref=ag_own.at[:, :],
                dst_ref=ag_recv.at[off - 1],
                send_sem=ag_send_sems.at[off - 1],
                recv_sem=ag_recv_sems.at[off - 1],
                device_id=(p,),
                device_id_type=_MESH)
            rd.start()
            ag_rdmas.append(rd)
        out_ref[0, pl.ds(my * QROWS, QROWS), :] = acc
        for rd in ag_rdmas:
            rd.wait_recv()
        for off in (1, 2, 3):
            j = lax.rem(my + (N_DEV - off), N_DEV)
            out_ref[0, pl.ds(j * QROWS, QROWS), :] = (
                ag_recv[off - 1].astype(jnp.float32))

        for rd in kv_rdmas + rs_rdmas + ag_rdmas:
            rd.wait_send()

    bf = jnp.bfloat16
    f32 = jnp.float32
    return pl.pallas_call(
        body,
        out_shape=jax.ShapeDtypeStruct((1, SQ, DM), f32),
        in_specs=[pl.BlockSpec(memory_space=_ANY)] * 5,
        out_specs=pl.BlockSpec(memory_space=pltpu.VMEM),
        scratch_shapes=[
            pltpu.VMEM((2, NB, GKV, H_TOT, DH), bf),
            pltpu.VMEM((NB, N_DEV * GKV, H_LOC, DH), bf),
            pltpu.VMEM((NB, N_DEV * GKV, H_LOC, DH), bf),
            pltpu.VMEM((2, BLK, H_TOT, DH), f32),
            pltpu.VMEM((2, 256, DM), f32),
            pltpu.VMEM((DM, DM), bf),
            pltpu.VMEM((DM, DM), bf),
            pltpu.VMEM((DM, DM), bf),
            pltpu.VMEM((SQ, H_LOC * DH), bf),
            pltpu.VMEM((SQ, H_LOC * DH), bf),
            pltpu.VMEM((SQ, DM), bf),
            pltpu.VMEM((3, QROWS, DM), bf),
            pltpu.VMEM((QROWS, DM), bf),
            pltpu.VMEM((3, QROWS, DM), bf),
            pltpu.SemaphoreType.DMA((2,)),
            pltpu.SemaphoreType.DMA((2,)),
            pltpu.SemaphoreType.DMA((2,)),
            pltpu.SemaphoreType.DMA((3,)),
            pltpu.SemaphoreType.DMA((3,)),
            pltpu.SemaphoreType.DMA((3,)),
            pltpu.SemaphoreType.DMA((3,)),
            pltpu.SemaphoreType.DMA((3,)),
            pltpu.SemaphoreType.DMA((3,)),
            pltpu.SemaphoreType.DMA((3,)),
            pltpu.SemaphoreType.DMA((3,)),
        ],
        compiler_params=_Params(collective_id=0,
                                vmem_limit_bytes=63 * 1024 * 1024),
    )(x, Wq, K_ext, V_ext, Wo)


# device time: 147945 ns/iter; 1.3615x vs baseline; 1.3615x over previous
import jax
import jax.numpy as jnp
from jax import lax
from jax.experimental import pallas as pl
from jax.experimental.pallas import tpu as pltpu

N_DEV = 4
SQ = 1024
SKV_LOC = 1024
H_LOC = 8
H_TOT = 32
DH = 128
DM = 1024
NB = 4
BLK = 64
GKV = SKV_LOC // NB
QROWS = SQ // N_DEV
SCALE = 0.08838834764831843

_MESH = pl.DeviceIdType.MESH if hasattr(pl, "DeviceIdType") else pltpu.DeviceIdType.MESH
_sem_signal = pl.semaphore_signal if hasattr(pl, "semaphore_signal") else pltpu.semaphore_signal
_sem_wait = pl.semaphore_wait if hasattr(pl, "semaphore_wait") else pltpu.semaphore_wait
_Params = getattr(pltpu, "CompilerParams", None) or getattr(pltpu, "TPUCompilerParams")
_ANY = pl.ANY


def kernel(x, Wq, K_ext, V_ext, Wo):
    def body(x_hbm, wq_hbm, k_hbm, v_hbm, wo_hbm, out_ref,
             kv_send, k_full, v_full, stage, wstage,
             x_bf, wq_bf, wo_bf, q_bf, ctx_buf, p_send,
             rs_recv, ag_own, ag_recv,
             stage_sems, wstage_sems, own_sems,
             k_send_sems, k_recv_sems, v_send_sems, v_recv_sems,
             rs_send_sems, rs_recv_sems, ag_send_sems, ag_recv_sems):
        my = lax.axis_index("i")

        bar = pltpu.get_barrier_semaphore()
        for off in (1, 2, 3):
            _sem_signal(bar, inc=1,
                        device_id=(lax.rem(my + off, N_DEV),),
                        device_id_type=_MESH)
        _sem_wait(bar, N_DEV - 1)

        items = [(t, 4 * n + r) for r in range(NB)
                 for t in (0, 1) for n in range(4)]

        def _kv_src(t, b):
            ref = k_hbm if t == 0 else v_hbm
            return ref.at[0, pl.ds(b * BLK, BLK), :, :]

        kv_rdmas = []
        grp_rdmas = [[] for _ in range(NB)]
        own_cps = []

        def _send_group(r):
            ko = pltpu.make_async_copy(
                kv_send.at[0, r, :, pl.ds(my * H_LOC, H_LOC), :],
                k_full.at[r, 3 * GKV:4 * GKV, :, :], own_sems.at[0, r])
            vo = pltpu.make_async_copy(
                kv_send.at[1, r, :, pl.ds(my * H_LOC, H_LOC), :],
                v_full.at[r, 3 * GKV:4 * GKV, :, :], own_sems.at[1, r])
            ko.start()
            vo.start()
            own_cps.append((ko, vo))
            for off in (1, 2, 3):
                p = lax.rem(my + off, N_DEV)
                lo, hi = (off - 1) * GKV, off * GKV
                for t, full, ssem, rsem in (
                        (0, k_full, k_send_sems, k_recv_sems),
                        (1, v_full, v_send_sems, v_recv_sems)):
                    rd = pltpu.make_async_remote_copy(
                        src_ref=kv_send.at[t, r, :, pl.ds(p * H_LOC, H_LOC), :],
                        dst_ref=full.at[r, lo:hi, :, :],
                        send_sem=ssem.at[off - 1, r],
                        recv_sem=rsem.at[off - 1, r],
                        device_id=(p,),
                        device_id_type=_MESH)
                    rd.start()
                    kv_rdmas.append(rd)
                    grp_rdmas[r].append(rd)

        cps = [None, None]
        cps[0] = pltpu.make_async_copy(_kv_src(*items[0]), stage.at[0],
                                       stage_sems.at[0])
        cps[0].start()
        for i, (t, b) in enumerate(items):
            slot = i % 2
            if i + 1 < len(items):
                ns = (i + 1) % 2
                cps[ns] = pltpu.make_async_copy(_kv_src(*items[i + 1]),
                                                stage.at[ns], stage_sems.at[ns])
                cps[ns].start()
            cps[slot].wait()
            r, n = b % NB, b // NB
            kv_send[t, r, n * BLK:(n + 1) * BLK, :, :] = (
                stage[slot].astype(jnp.bfloat16))
            if i % 8 == 7:
                _send_group(i // 8)

        mats = [(x_hbm.at[0], x_bf), (wq_hbm.at[:, :], wq_bf),
                (wo_hbm.at[:, :], wo_bf)]
        witems = [(mi, c) for mi in range(3) for c in range(4)]
        wps = [None, None]
        wps[0] = pltpu.make_async_copy(
            mats[0][0].at[pl.ds(0, 256), :], wstage.at[0], wstage_sems.at[0])
        wps[0].start()
        for i, (mi, c) in enumerate(witems):
            slot = i % 2
            if i + 1 < len(witems):
                nmi, nc = witems[i + 1]
                ns = (i + 1) % 2
                wps[ns] = pltpu.make_async_copy(
                    mats[nmi][0].at[pl.ds(nc * 256, 256), :],
                    wstage.at[ns], wstage_sems.at[ns])
                wps[ns].start()
            wps[slot].wait()
            mats[mi][1][c * 256:(c + 1) * 256, :] = (
                wstage[slot].astype(jnp.bfloat16))

        for c in range(4):
            qc = jnp.dot(x_bf[c * 256:(c + 1) * 256, :], wq_bf[:, :],
                         preferred_element_type=jnp.float32)
            q_bf[c * 256:(c + 1) * 256, :] = qc.astype(jnp.bfloat16)

        for r in range(NB):
            ko, vo = own_cps[r]
            ko.wait()
            vo.wait()
            for rd in grp_rdmas[r]:
                rd.wait_recv()
            for h in range(H_LOC):
                q_rh = jnp.concatenate(
                    [q_bf[(NB * m + r) * BLK:(NB * m + r + 1) * BLK,
                          h * DH:(h + 1) * DH] for m in range(4)], axis=0)
                k_rh = k_full[r, :, h, :]
                s = lax.dot_general(
                    q_rh, k_rh, (((1,), (1,)), ((), ())),
                    preferred_element_type=jnp.float32)
                s = s * SCALE
                mx = jnp.max(s, axis=1, keepdims=True)
                w = jnp.exp(s - mx)
                den = jnp.sum(w, axis=1, keepdims=True)
                pw = (w / den).astype(jnp.bfloat16)
                v_rh = v_full[r, :, h, :]
                ctx = jnp.dot(pw, v_rh,
                              preferred_element_type=jnp.float32)
                ctxb = ctx.astype(jnp.bfloat16)
                for m in range(4):
                    ctx_buf[(NB * m + r) * BLK:(NB * m + r + 1) * BLK,
                            h * DH:(h + 1) * DH] = ctxb[m * BLK:(m + 1) * BLK]

        rs_rdmas = []
        for off in (1, 2, 3):
            p = lax.rem(my + off, N_DEV)
            pc = jnp.dot(ctx_buf[pl.ds(p * QROWS, QROWS), :], wo_bf[:, :],
                         preferred_element_type=jnp.float32)
            p_send[pl.ds(p * QROWS, QROWS), :] = pc.astype(jnp.bfloat16)
            rd = pltpu.make_async_remote_copy(
                src_ref=p_send.at[pl.ds(p * QROWS, QROWS), :],
                dst_ref=rs_recv.at[off - 1],
                send_sem=rs_send_sems.at[off - 1],
                recv_sem=rs_recv_sems.at[off - 1],
                device_id=(p,),
                device_id_type=_MESH)
            rd.start()
            rs_rdmas.append(rd)
        acc = jnp.dot(ctx_buf[pl.ds(my * QROWS, QROWS), :], wo_bf[:, :],
                      preferred_element_type=jnp.float32)
        for o, rd in enumerate(rs_rdmas):
            rd.wait_recv()
            acc = acc + rs_recv[o].astype(jnp.float32)
        ag_own[:, :] = acc.astype(jnp.bfloat16)

        ag_rdmas = []
        for off in (1, 2, 3):
            p = lax.rem(my + off, N_DEV)
            rd = pltpu.make_async_remote_copy(
                src_ref=ag_own.at[:, :],
                dst_ref=ag_recv.at[off - 1],
                send_sem=ag_send_sems.at[off - 1],
                recv_sem=ag_recv_sems.at[off - 1],
                device_id=(p,),
                device_id_type=_MESH)
            rd.start()
            ag_rdmas.append(rd)
        out_ref[0, pl.ds(my * QROWS, QROWS), :] = acc
        for rd in ag_rdmas:
            rd.wait_recv()
        for off in (1, 2, 3):
            j = lax.rem(my + (N_DEV - off), N_DEV)
            out_ref[0, pl.ds(j * QROWS, QROWS), :] = (
                ag_recv[off - 1].astype(jnp.float32))

        for rd in kv_rdmas + rs_rdmas + ag_rdmas:
            rd.wait_send()

    bf = jnp.bfloat16
    f32 = jnp.float32
    return pl.pallas_call(
        body,
        out_shape=jax.ShapeDtypeStruct((1, SQ, DM), f32),
        in_specs=[pl.BlockSpec(memory_space=_ANY)] * 5,
        out_specs=pl.BlockSpec(memory_space=pltpu.VMEM),
        scratch_shapes=[
            pltpu.VMEM((2, NB, GKV, H_TOT, DH), bf),
            pltpu.VMEM((NB, N_DEV * GKV, H_LOC, DH), bf),
            pltpu.VMEM((NB, N_DEV * GKV, H_LOC, DH), bf),
            pltpu.VMEM((2, BLK, H_TOT, DH), f32),
            pltpu.VMEM((2, 256, DM), f32),
            pltpu.VMEM((DM, DM), bf),
            pltpu.VMEM((DM, DM), bf),
            pltpu.VMEM((DM, DM), bf),
            pltpu.VMEM((SQ, H_LOC * DH), bf),
            pltpu.VMEM((SQ, H_LOC * DH), bf),
            pltpu.VMEM((SQ, DM), bf),
            pltpu.VMEM((3, QROWS, DM), bf),
            pltpu.VMEM((QROWS, DM), bf),
            pltpu.VMEM((3, QROWS, DM), bf),
            pltpu.SemaphoreType.DMA((2,)),
            pltpu.SemaphoreType.DMA((2,)),
            pltpu.SemaphoreType.DMA((2, NB)),
            pltpu.SemaphoreType.DMA((3, NB)),
            pltpu.SemaphoreType.DMA((3, NB)),
            pltpu.SemaphoreType.DMA((3, NB)),
            pltpu.SemaphoreType.DMA((3, NB)),
            pltpu.SemaphoreType.DMA((3,)),
            pltpu.SemaphoreType.DMA((3,)),
            pltpu.SemaphoreType.DMA((3,)),
            pltpu.SemaphoreType.DMA((3,)),
        ],
        compiler_params=_Params(collective_id=0,
                                vmem_limit_bytes=63 * 1024 * 1024),
    )(x, Wq, K_ext, V_ext, Wo)


# device time: 128964 ns/iter; 1.5619x vs baseline; 1.1472x over previous
import jax
import jax.numpy as jnp
from jax import lax
from jax.experimental import pallas as pl
from jax.experimental.pallas import tpu as pltpu

N_DEV = 4
SQ = 1024
SKV_LOC = 1024
H_LOC = 8
H_TOT = 32
DH = 128
DM = 1024
NB = 4
BLK = 64
GKV = SKV_LOC // NB
QROWS = SQ // N_DEV
SCALE = 0.08838834764831843

_MESH = pl.DeviceIdType.MESH if hasattr(pl, "DeviceIdType") else pltpu.DeviceIdType.MESH
_sem_signal = pl.semaphore_signal if hasattr(pl, "semaphore_signal") else pltpu.semaphore_signal
_sem_wait = pl.semaphore_wait if hasattr(pl, "semaphore_wait") else pltpu.semaphore_wait
_Params = getattr(pltpu, "CompilerParams", None) or getattr(pltpu, "TPUCompilerParams")
_ANY = pl.ANY


def kernel(x, Wq, K_ext, V_ext, Wo):
    def body(x_hbm, wq_hbm, k_hbm, v_hbm, wo_hbm, out_ref,
             kv_send, kv_send8, k_full, v_full, kv8_recv, stage, wstage,
             x_bf, wq_bf, wo_bf, q_bf, ctx_buf, p_send,
             rs_recv, ag_own, ag_recv,
             stage_sems, wstage_sems, own_sems,
             k_send_sems, k_recv_sems, v_send_sems, v_recv_sems,
             rs_send_sems, rs_recv_sems, ag_send_sems, ag_recv_sems):
        my = lax.axis_index("i")

        bar = pltpu.get_barrier_semaphore()
        for off in (1, 2, 3):
            _sem_signal(bar, inc=1,
                        device_id=(lax.rem(my + off, N_DEV),),
                        device_id_type=_MESH)
        _sem_wait(bar, N_DEV - 1)

        items = [(t, 4 * n + r) for r in range(NB)
                 for t in (0, 1) for n in range(4)]

        def _kv_src(t, b):
            ref = k_hbm if t == 0 else v_hbm
            return ref.at[0, pl.ds(b * BLK, BLK), :, :]

        kv_rdmas = []
        grp_rdmas = [[] for _ in range(NB)]
        own_cps = []

        def _send_group(r):
            ko = pltpu.make_async_copy(
                kv_send.at[0, r, :, pl.ds(my * H_LOC, H_LOC), :],
                k_full.at[r, 3 * GKV:4 * GKV, :, :], own_sems.at[0, r])
            vo = pltpu.make_async_copy(
                kv_send.at[1, r, :, pl.ds(my * H_LOC, H_LOC), :],
                v_full.at[r, 3 * GKV:4 * GKV, :, :], own_sems.at[1, r])
            ko.start()
            vo.start()
            own_cps.append((ko, vo))
            p2 = lax.rem(my + 2, N_DEV)
            for t in (0, 1):
                kv_send8[t, r] = (
                    kv_send[t, r, :, pl.ds(p2 * H_LOC, H_LOC), :]
                    .astype(jnp.float8_e4m3fn))
            for off in (1, 2, 3):
                p = lax.rem(my + off, N_DEV)
                lo, hi = (off - 1) * GKV, off * GKV
                for t, full, ssem, rsem in (
                        (0, k_full, k_send_sems, k_recv_sems),
                        (1, v_full, v_send_sems, v_recv_sems)):
                    if off == 2:
                        rd = pltpu.make_async_remote_copy(
                            src_ref=kv_send8.at[t, r],
                            dst_ref=kv8_recv.at[t, r],
                            send_sem=ssem.at[off - 1, r],
                            recv_sem=rsem.at[off - 1, r],
                            device_id=(p,),
                            device_id_type=_MESH)
                    else:
                        rd = pltpu.make_async_remote_copy(
                            src_ref=kv_send.at[t, r, :,
                                               pl.ds(p * H_LOC, H_LOC), :],
                            dst_ref=full.at[r, lo:hi, :, :],
                            send_sem=ssem.at[off - 1, r],
                            recv_sem=rsem.at[off - 1, r],
                            device_id=(p,),
                            device_id_type=_MESH)
                    rd.start()
                    kv_rdmas.append(rd)
                    grp_rdmas[r].append(rd)

        cps = [None, None]
        cps[0] = pltpu.make_async_copy(_kv_src(*items[0]), stage.at[0],
                                       stage_sems.at[0])
        cps[0].start()
        for i, (t, b) in enumerate(items):
            slot = i % 2
            if i + 1 < len(items):
                ns = (i + 1) % 2
                cps[ns] = pltpu.make_async_copy(_kv_src(*items[i + 1]),
                                                stage.at[ns], stage_sems.at[ns])
                cps[ns].start()
            cps[slot].wait()
            r, n = b % NB, b // NB
            kv_send[t, r, n * BLK:(n + 1) * BLK, :, :] = (
                stage[slot].astype(jnp.bfloat16))
            if i % 8 == 7:
                _send_group(i // 8)

        mats = [(x_hbm.at[0], x_bf), (wq_hbm.at[:, :], wq_bf),
                (wo_hbm.at[:, :], wo_bf)]
        witems = [(mi, c) for mi in range(3) for c in range(4)]
        for mi, c in witems:
            cp = pltpu.make_async_copy(
                mats[mi][0].at[pl.ds(c * 256, 256), :],
                wstage.at[0], wstage_sems.at[0])
            cp.start()
            cp.wait()
            mats[mi][1][c * 256:(c + 1) * 256, :] = (
                wstage[0].astype(jnp.bfloat16))

        for c in range(4):
            qc = jnp.dot(x_bf[c * 256:(c + 1) * 256, :], wq_bf[:, :],
                         preferred_element_type=jnp.float32)
            q_bf[c * 256:(c + 1) * 256, :] = qc.astype(jnp.bfloat16)

        for r in range(NB):
            ko, vo = own_cps[r]
            ko.wait()
            vo.wait()
            for rd in grp_rdmas[r]:
                rd.wait_recv()
            k_full[r, GKV:2 * GKV, :, :] = kv8_recv[0, r].astype(jnp.bfloat16)
            v_full[r, GKV:2 * GKV, :, :] = kv8_recv[1, r].astype(jnp.bfloat16)
            for h in range(H_LOC):
                q_rh = jnp.concatenate(
                    [q_bf[(NB * m + r) * BLK:(NB * m + r + 1) * BLK,
                          h * DH:(h + 1) * DH] for m in range(4)], axis=0)
                k_rh = k_full[r, :, h, :]
                s = lax.dot_general(
                    q_rh, k_rh, (((1,), (1,)), ((), ())),
                    preferred_element_type=jnp.float32)
                s = s * SCALE
                mx = jnp.max(s, axis=1, keepdims=True)
                w = jnp.exp(s - mx)
                den = jnp.sum(w, axis=1, keepdims=True)
                pw = (w / den).astype(jnp.bfloat16)
                v_rh = v_full[r, :, h, :]
                ctx = jnp.dot(pw, v_rh,
                              preferred_element_type=jnp.float32)
                ctxb = ctx.astype(jnp.bfloat16)
                for m in range(4):
                    ctx_buf[(NB * m + r) * BLK:(NB * m + r + 1) * BLK,
                            h * DH:(h + 1) * DH] = ctxb[m * BLK:(m + 1) * BLK]

        rs_rdmas = []
        for off in (1, 2, 3):
            p = lax.rem(my + off, N_DEV)
            pc = jnp.dot(ctx_buf[pl.ds(p * QROWS, QROWS), :], wo_bf[:, :],
                         preferred_element_type=jnp.float32)
            p_send[pl.ds(p * QROWS, QROWS), :] = pc.astype(jnp.bfloat16)
            rd = pltpu.make_async_remote_copy(
                src_ref=p_send.at[pl.ds(p * QROWS, QROWS), :],
                dst_ref=rs_recv.at[off - 1],
                send_sem=rs_send_sems.at[off - 1],
                recv_sem=rs_recv_sems.at[off - 1],
                device_id=(p,),
                device_id_type=_MESH)
            rd.start()
            rs_rdmas.append(rd)
        acc = jnp.dot(ctx_buf[pl.ds(my * QROWS, QROWS), :], wo_bf[:, :],
                      preferred_element_type=jnp.float32)
        for o, rd in enumerate(rs_rdmas):
            rd.wait_recv()
            acc = acc + rs_recv[o].astype(jnp.float32)
        ag_own[:, :] = acc.astype(jnp.bfloat16)

        ag_rdmas = []
        for off in (1, 2, 3):
            p = lax.rem(my + off, N_DEV)
            rd = pltpu.make_async_remote_copy(
                src_ref=ag_own.at[:, :],
                dst_ref=ag_recv.at[off - 1],
                send_sem=ag_send_sems.at[off - 1],
                recv_sem=ag_recv_sems.at[off - 1],
                device_id=(p,),
                device_id_type=_MESH)
            rd.start()
            ag_rdmas.append(rd)
        out_ref[0, pl.ds(my * QROWS, QROWS), :] = acc
        for rd in ag_rdmas:
            rd.wait_recv()
        for off in (1, 2, 3):
            j = lax.rem(my + (N_DEV - off), N_DEV)
            out_ref[0, pl.ds(j * QROWS, QROWS), :] = (
                ag_recv[off - 1].astype(jnp.float32))

        for rd in kv_rdmas + rs_rdmas + ag_rdmas:
            rd.wait_send()

    bf = jnp.bfloat16
    f32 = jnp.float32
    return pl.pallas_call(
        body,
        out_shape=jax.ShapeDtypeStruct((1, SQ, DM), f32),
        in_specs=[pl.BlockSpec(memory_space=_ANY)] * 5,
        out_specs=pl.BlockSpec(memory_space=pltpu.VMEM),
        scratch_shapes=[
            pltpu.VMEM((2, NB, GKV, H_TOT, DH), bf),
            pltpu.VMEM((2, NB, GKV, H_LOC, DH), jnp.float8_e4m3fn),
            pltpu.VMEM((NB, N_DEV * GKV, H_LOC, DH), bf),
            pltpu.VMEM((NB, N_DEV * GKV, H_LOC, DH), bf),
            pltpu.VMEM((2, NB, GKV, H_LOC, DH), jnp.float8_e4m3fn),
            pltpu.VMEM((2, BLK, H_TOT, DH), f32),
            pltpu.VMEM((1, 256, DM), f32),
            pltpu.VMEM((DM, DM), bf),
            pltpu.VMEM((DM, DM), bf),
            pltpu.VMEM((DM, DM), bf),
            pltpu.VMEM((SQ, H_LOC * DH), bf),
            pltpu.VMEM((SQ, H_LOC * DH), bf),
            pltpu.VMEM((SQ, DM), bf),
            pltpu.VMEM((3, QROWS, DM), bf),
            pltpu.VMEM((QROWS, DM), bf),
            pltpu.VMEM((3, QROWS, DM), bf),
            pltpu.SemaphoreType.DMA((2,)),
            pltpu.SemaphoreType.DMA((1,)),
            pltpu.SemaphoreType.DMA((2, NB)),
            pltpu.SemaphoreType.DMA((3, NB)),
            pltpu.SemaphoreType.DMA((3, NB)),
            pltpu.SemaphoreType.DMA((3, NB)),
            pltpu.SemaphoreType.DMA((3, NB)),
            pltpu.SemaphoreType.DMA((3,)),
            pltpu.SemaphoreType.DMA((3,)),
            pltpu.SemaphoreType.DMA((3,)),
            pltpu.SemaphoreType.DMA((3,)),
        ],
        compiler_params=_Params(collective_id=0,
                                vmem_limit_bytes=67000000),
    )(x, Wq, K_ext, V_ext, Wo)


# device time: 126524 ns/iter; 1.5921x vs baseline; 1.0193x over previous
import jax
import jax.numpy as jnp
from jax import lax
from jax.experimental import pallas as pl
from jax.experimental.pallas import tpu as pltpu

N_DEV = 4
SQ = 1024
SKV_LOC = 1024
H_LOC = 8
H_TOT = 32
DH = 128
DM = 1024
NB = 4
BLK = 64
GKV = SKV_LOC // NB
QROWS = SQ // N_DEV
SCALE = 0.08838834764831843

_MESH = pl.DeviceIdType.MESH if hasattr(pl, "DeviceIdType") else pltpu.DeviceIdType.MESH
_sem_signal = pl.semaphore_signal if hasattr(pl, "semaphore_signal") else pltpu.semaphore_signal
_sem_wait = pl.semaphore_wait if hasattr(pl, "semaphore_wait") else pltpu.semaphore_wait
_Params = getattr(pltpu, "CompilerParams", None) or getattr(pltpu, "TPUCompilerParams")
_ANY = pl.ANY


def kernel(x, Wq, K_ext, V_ext, Wo):
    def body(x_hbm, wq_hbm, k_hbm, v_hbm, wo_hbm, out_ref,
             kv_send, kv_send8, k_full, v_full, kv8_recv, stage, wstage,
             x_bf, wq_bf, wo_bf, q_bf, ctx_buf, p_send,
             rs_recv, ag_own, ag_recv,
             stage_sems, wstage_sems, own_sems,
             k_send_sems, k_recv_sems, v_send_sems, v_recv_sems,
             rs_send_sems, rs_recv_sems, ag_send_sems, ag_recv_sems):
        my = lax.axis_index("i")

        bar = pltpu.get_barrier_semaphore()
        for off in (1, 2, 3):
            _sem_signal(bar, inc=1,
                        device_id=(lax.rem(my + off, N_DEV),),
                        device_id_type=_MESH)
        _sem_wait(bar, N_DEV - 1)

        items = [(t, 4 * n + r) for r in range(NB)
                 for t in (0, 1) for n in range(4)]

        def _kv_src(t, b):
            ref = k_hbm if t == 0 else v_hbm
            return ref.at[0, pl.ds(b * BLK, BLK), :, :]

        kv_rdmas = []
        grp_rdmas = [[] for _ in range(NB)]
        own_cps = []

        def _send_group(r):
            ko = pltpu.make_async_copy(
                kv_send.at[0, r, :, pl.ds(my * H_LOC, H_LOC), :],
                k_full.at[r, 3 * GKV:4 * GKV, :, :], own_sems.at[0, r])
            vo = pltpu.make_async_copy(
                kv_send.at[1, r, :, pl.ds(my * H_LOC, H_LOC), :],
                v_full.at[r, 3 * GKV:4 * GKV, :, :], own_sems.at[1, r])
            ko.start()
            vo.start()
            own_cps.append((ko, vo))
            p2 = lax.rem(my + 2, N_DEV)
            for t in (0, 1):
                kv_send8[t, r] = (
                    kv_send[t, r, :, pl.ds(p2 * H_LOC, H_LOC), :]
                    .astype(jnp.float8_e4m3fn))
            for off in (1, 2, 3):
                p = lax.rem(my + off, N_DEV)
                lo, hi = (off - 1) * GKV, off * GKV
                for t, full, ssem, rsem in (
                        (0, k_full, k_send_sems, k_recv_sems),
                        (1, v_full, v_send_sems, v_recv_sems)):
                    if off == 2:
                        rd = pltpu.make_async_remote_copy(
                            src_ref=kv_send8.at[t, r],
                            dst_ref=kv8_recv.at[t, r],
                            send_sem=ssem.at[off - 1, r],
                            recv_sem=rsem.at[off - 1, r],
                            device_id=(p,),
                            device_id_type=_MESH)
                    else:
                        rd = pltpu.make_async_remote_copy(
                            src_ref=kv_send.at[t, r, :,
                                               pl.ds(p * H_LOC, H_LOC), :],
                            dst_ref=full.at[r, lo:hi, :, :],
                            send_sem=ssem.at[off - 1, r],
                            recv_sem=rsem.at[off - 1, r],
                            device_id=(p,),
                            device_id_type=_MESH)
                    rd.start()
                    kv_rdmas.append(rd)
                    grp_rdmas[r].append(rd)

        cps = [None, None]
        cps[0] = pltpu.make_async_copy(_kv_src(*items[0]), stage.at[0],
                                       stage_sems.at[0])
        cps[0].start()
        for i, (t, b) in enumerate(items):
            slot = i % 2
            if i + 1 < len(items):
                ns = (i + 1) % 2
                cps[ns] = pltpu.make_async_copy(_kv_src(*items[i + 1]),
                                                stage.at[ns], stage_sems.at[ns])
                cps[ns].start()
            cps[slot].wait()
            r, n = b % NB, b // NB
            kv_send[t, r, n * BLK:(n + 1) * BLK, :, :] = (
                stage[slot].astype(jnp.bfloat16))
            if i % 8 == 7:
                _send_group(i // 8)

        mats = [(x_hbm.at[0], x_bf), (wq_hbm.at[:, :], wq_bf),
                (wo_hbm.at[:, :], wo_bf)]
        witems = [(mi, c) for mi in range(3) for c in range(4)]
        for mi, c in witems:
            cp = pltpu.make_async_copy(
                mats[mi][0].at[pl.ds(c * 256, 256), :],
                wstage.at[0], wstage_sems.at[0])
            cp.start()
            cp.wait()
            mats[mi][1][c * 256:(c + 1) * 256, :] = (
                wstage[0].astype(jnp.bfloat16))

        for c in range(4):
            qc = jnp.dot(x_bf[c * 256:(c + 1) * 256, :], wq_bf[:, :],
                         preferred_element_type=jnp.float32)
            q_bf[c * 256:(c + 1) * 256, :] = (qc * SCALE).astype(jnp.bfloat16)

        for r in range(NB):
            ko, vo = own_cps[r]
            ko.wait()
            vo.wait()
            for rd in grp_rdmas[r]:
                rd.wait_recv()
            k_full[r, GKV:2 * GKV, :, :] = kv8_recv[0, r].astype(jnp.bfloat16)
            v_full[r, GKV:2 * GKV, :, :] = kv8_recv[1, r].astype(jnp.bfloat16)
            for h in range(H_LOC):
                q_rh = jnp.concatenate(
                    [q_bf[(NB * m + r) * BLK:(NB * m + r + 1) * BLK,
                          h * DH:(h + 1) * DH] for m in range(4)], axis=0)
                k_rh = k_full[r, :, h, :]
                s = lax.dot_general(
                    q_rh, k_rh, (((1,), (1,)), ((), ())),
                    preferred_element_type=jnp.float32)
                mx = jnp.max(s, axis=1, keepdims=True)
                w = jnp.exp(s - mx)
                den = jnp.sum(w, axis=1, keepdims=True)
                pw = (w * (1.0 / den)).astype(jnp.bfloat16)
                v_rh = v_full[r, :, h, :]
                ctx = jnp.dot(pw, v_rh,
                              preferred_element_type=jnp.float32)
                ctxb = ctx.astype(jnp.bfloat16)
                for m in range(4):
                    ctx_buf[(NB * m + r) * BLK:(NB * m + r + 1) * BLK,
                            h * DH:(h + 1) * DH] = ctxb[m * BLK:(m + 1) * BLK]

        HD = DM // 2
        rs_rdmas = []
        for off in (1, 2, 3):
            p = lax.rem(my + off, N_DEV)
            pc = jnp.dot(ctx_buf[pl.ds(p * QROWS, QROWS), :], wo_bf[:, :],
                         preferred_element_type=jnp.float32)
            p_send[pl.ds(p * QROWS, QROWS), :] = pc.astype(jnp.bfloat16)
            for hf in (0, 1):
                rd = pltpu.make_async_remote_copy(
                    src_ref=p_send.at[pl.ds(p * QROWS, QROWS),
                                      hf * HD:(hf + 1) * HD],
                    dst_ref=rs_recv.at[off - 1, :, hf * HD:(hf + 1) * HD],
                    send_sem=rs_send_sems.at[off - 1, hf],
                    recv_sem=rs_recv_sems.at[off - 1, hf],
                    device_id=(p,),
                    device_id_type=_MESH)
                rd.start()
                rs_rdmas.append((hf, rd))
        acc_own = jnp.dot(ctx_buf[pl.ds(my * QROWS, QROWS), :], wo_bf[:, :],
                          preferred_element_type=jnp.float32)

        ag_rdmas = []
        accs = []
        for hf in (0, 1):
            for h, rd in rs_rdmas:
                if h == hf:
                    rd.wait_recv()
            acc_h = acc_own[:, hf * HD:(hf + 1) * HD]
            for o in range(3):
                acc_h = acc_h + rs_recv[o, :, hf * HD:(hf + 1) * HD].astype(
                    jnp.float32)
            accs.append(acc_h)
            ag_own[:, hf * HD:(hf + 1) * HD] = acc_h.astype(jnp.bfloat16)
            for off in (1, 2, 3):
                p = lax.rem(my + off, N_DEV)
                rd = pltpu.make_async_remote_copy(
                    src_ref=ag_own.at[:, hf * HD:(hf + 1) * HD],
                    dst_ref=ag_recv.at[off - 1, :, hf * HD:(hf + 1) * HD],
                    send_sem=ag_send_sems.at[off - 1, hf],
                    recv_sem=ag_recv_sems.at[off - 1, hf],
                    device_id=(p,),
                    device_id_type=_MESH)
                rd.start()
                ag_rdmas.append(rd)
        out_ref[0, pl.ds(my * QROWS, QROWS), :] = jnp.concatenate(accs, axis=1)
        for rd in ag_rdmas:
            rd.wait_recv()
        for off in (1, 2, 3):
            j = lax.rem(my + (N_DEV - off), N_DEV)
            out_ref[0, pl.ds(j * QROWS, QROWS), :] = (
                ag_recv[off - 1].astype(jnp.float32))

        for rd in kv_rdmas + ag_rdmas:
            rd.wait_send()
        for _, rd in rs_rdmas:
            rd.wait_send()

    bf = jnp.bfloat16
    f32 = jnp.float32
    return pl.pallas_call(
        body,
        out_shape=jax.ShapeDtypeStruct((1, SQ, DM), f32),
        in_specs=[pl.BlockSpec(memory_space=_ANY)] * 5,
        out_specs=pl.BlockSpec(memory_space=pltpu.VMEM),
        scratch_shapes=[
            pltpu.VMEM((2, NB, GKV, H_TOT, DH), bf),
            pltpu.VMEM((2, NB, GKV, H_LOC, DH), jnp.float8_e4m3fn),
            pltpu.VMEM((NB, N_DEV * GKV, H_LOC, DH), bf),
            pltpu.VMEM((NB, N_DEV * GKV, H_LOC, DH), bf),
            pltpu.VMEM((2, NB, GKV, H_LOC, DH), jnp.float8_e4m3fn),
            pltpu.VMEM((2, BLK, H_TOT, DH), f32),
            pltpu.VMEM((1, 256, DM), f32),
            pltpu.VMEM((DM, DM), bf),
            pltpu.VMEM((DM, DM), bf),
            pltpu.VMEM((DM, DM), bf),
            pltpu.VMEM((SQ, H_LOC * DH), bf),
            pltpu.VMEM((SQ, H_LOC * DH), bf),
            pltpu.VMEM((SQ, DM), bf),
            pltpu.VMEM((3, QROWS, DM), bf),
            pltpu.VMEM((QROWS, DM), bf),
            pltpu.VMEM((3, QROWS, DM), bf),
            pltpu.SemaphoreType.DMA((2,)),
            pltpu.SemaphoreType.DMA((1,)),
            pltpu.SemaphoreType.DMA((2, NB)),
            pltpu.SemaphoreType.DMA((3, NB)),
            pltpu.SemaphoreType.DMA((3, NB)),
            pltpu.SemaphoreType.DMA((3, NB)),
            pltpu.SemaphoreType.DMA((3, NB)),
            pltpu.SemaphoreType.DMA((3, 2)),
            pltpu.SemaphoreType.DMA((3, 2)),
            pltpu.SemaphoreType.DMA((3, 2)),
            pltpu.SemaphoreType.DMA((3, 2)),
        ],
        compiler_params=_Params(collective_id=0,
                                vmem_limit_bytes=67000000),
    )(x, Wq, K_ext, V_ext, Wo)


# device time: 115148 ns/iter; 1.7493x vs baseline; 1.0988x over previous
import jax
import jax.numpy as jnp
from jax import lax
from jax.experimental import pallas as pl
from jax.experimental.pallas import tpu as pltpu

N_DEV = 4
SQ = 1024
SKV_LOC = 1024
H_LOC = 8
H_TOT = 32
DH = 128
DM = 1024
NB = 4
BLK = 64
GKV = SKV_LOC // NB
QROWS = SQ // N_DEV
SCALE = 0.08838834764831843

_MESH = pl.DeviceIdType.MESH if hasattr(pl, "DeviceIdType") else pltpu.DeviceIdType.MESH
_sem_signal = pl.semaphore_signal if hasattr(pl, "semaphore_signal") else pltpu.semaphore_signal
_sem_wait = pl.semaphore_wait if hasattr(pl, "semaphore_wait") else pltpu.semaphore_wait
_Params = getattr(pltpu, "CompilerParams", None) or getattr(pltpu, "TPUCompilerParams")
_ANY = pl.ANY


def kernel(x, Wq, K_ext, V_ext, Wo):
    def body(x_hbm, wq_hbm, k_hbm, v_hbm, wo_hbm, out_ref,
             kv_send, kv_send8, k_full, v_full, kv8_recv, stage, wstage,
             x_bf, wq_bf, wo_bf, q_bf, ctx_buf, p_send,
             rs_recv, ag_own, ag_recv,
             stage_sems, wstage_sems, own_sems,
             k_send_sems, k_recv_sems, v_send_sems, v_recv_sems,
             rs_send_sems, rs_recv_sems, ag_send_sems, ag_recv_sems):
        my = lax.axis_index("i")

        bar = pltpu.get_barrier_semaphore()
        for off in (1, 2, 3):
            _sem_signal(bar, inc=1,
                        device_id=(lax.rem(my + off, N_DEV),),
                        device_id_type=_MESH)

        items = [(t, 4 * n + r) for r in range(NB)
                 for t in (0, 1) for n in range(4)]

        def _start_block(i, slot):
            t, b = items[i]
            ref = k_hbm if t == 0 else v_hbm
            cps = []
            for h in range(H_TOT):
                cp = pltpu.make_async_copy(
                    ref.at[0, pl.ds(b * BLK, BLK), h, :],
                    stage.at[slot, h], stage_sems.at[slot])
                cp.start()
                cps.append(cp)
            return cps

        kv_rdmas = []
        grp_rdmas = [[] for _ in range(NB)]
        own_cps = []

        def _send_group(r):
            ko = pltpu.make_async_copy(
                kv_send.at[0, r, pl.ds(my * H_LOC, H_LOC), :, :],
                k_full.at[r, :, 3 * GKV:4 * GKV, :], own_sems.at[0, r])
            vo = pltpu.make_async_copy(
                kv_send.at[1, r, pl.ds(my * H_LOC, H_LOC), :, :],
                v_full.at[r, :, 3 * GKV:4 * GKV, :], own_sems.at[1, r])
            ko.start()
            vo.start()
            own_cps.append((ko, vo))
            p2 = lax.rem(my + 2, N_DEV)
            for t in (0, 1):
                kv_send8[t, r] = (
                    kv_send[t, r, pl.ds(p2 * H_LOC, H_LOC), :, :]
                    .astype(jnp.float8_e4m3fn))
            for off in (1, 2, 3):
                p = lax.rem(my + off, N_DEV)
                lo, hi = (off - 1) * GKV, off * GKV
                for t, full, ssem, rsem in (
                        (0, k_full, k_send_sems, k_recv_sems),
                        (1, v_full, v_send_sems, v_recv_sems)):
                    if off == 2:
                        rd = pltpu.make_async_remote_copy(
                            src_ref=kv_send8.at[t, r],
                            dst_ref=kv8_recv.at[t, r],
                            send_sem=ssem.at[off - 1, r],
                            recv_sem=rsem.at[off - 1, r],
                            device_id=(p,),
                            device_id_type=_MESH)
                    else:
                        rd = pltpu.make_async_remote_copy(
                            src_ref=kv_send.at[t, r,
                                               pl.ds(p * H_LOC, H_LOC), :, :],
                            dst_ref=full.at[r, :, lo:hi, :],
                            send_sem=ssem.at[off - 1, r],
                            recv_sem=rsem.at[off - 1, r],
                            device_id=(p,),
                            device_id_type=_MESH)
                    rd.start()
                    kv_rdmas.append(rd)
                    grp_rdmas[r].append(rd)

        pend = [None, None]
        pend[0] = _start_block(0, 0)
        for i, (t, b) in enumerate(items):
            slot = i % 2
            if i + 1 < len(items):
                pend[(i + 1) % 2] = _start_block(i + 1, (i + 1) % 2)
            for cp in pend[slot]:
                cp.wait()
            r, n = b % NB, b // NB
            kv_send[t, r, :, n * BLK:(n + 1) * BLK, :] = (
                stage[slot].astype(jnp.bfloat16))
            if i % 8 == 7:
                if i == 7:
                    _sem_wait(bar, N_DEV - 1)
                _send_group(i // 8)

        mats = [(x_hbm.at[0], x_bf), (wq_hbm.at[:, :], wq_bf),
                (wo_hbm.at[:, :], wo_bf)]
        witems = [(mi, c) for mi in range(3) for c in range(4)]
        for mi, c in witems:
            cp = pltpu.make_async_copy(
                mats[mi][0].at[pl.ds(c * 256, 256), :],
                wstage.at[0], wstage_sems.at[0])
            cp.start()
            cp.wait()
            mats[mi][1][c * 256:(c + 1) * 256, :] = (
                wstage[0].astype(jnp.bfloat16))

        for c in range(4):
            qc = jnp.dot(x_bf[c * 256:(c + 1) * 256, :], wq_bf[:, :],
                         preferred_element_type=jnp.float32)
            q_bf[c * 256:(c + 1) * 256, :] = (qc * SCALE).astype(jnp.bfloat16)

        for r in range(NB):
            ko, vo = own_cps[r]
            ko.wait()
            vo.wait()
            for rd in grp_rdmas[r]:
                rd.wait_recv()
            k_full[r, :, GKV:2 * GKV, :] = kv8_recv[0, r].astype(jnp.bfloat16)
            v_full[r, :, GKV:2 * GKV, :] = kv8_recv[1, r].astype(jnp.bfloat16)
            for h in range(H_LOC):
                q_rh = jnp.concatenate(
                    [q_bf[(NB * m + r) * BLK:(NB * m + r + 1) * BLK,
                          h * DH:(h + 1) * DH] for m in range(4)], axis=0)
                k_rh = k_full[r, h, :, :]
                s = lax.dot_general(
                    q_rh, k_rh, (((1,), (1,)), ((), ())),
                    preferred_element_type=jnp.float32)
                mx = jnp.max(s, axis=1, keepdims=True)
                w = jnp.exp(s - mx)
                den = jnp.sum(w, axis=1, keepdims=True)
                pw = (w * (1.0 / den)).astype(jnp.bfloat16)
                v_rh = v_full[r, h, :, :]
                ctx = jnp.dot(pw, v_rh,
                              preferred_element_type=jnp.float32)
                ctxb = ctx.astype(jnp.bfloat16)
                for m in range(4):
                    ctx_buf[(NB * m + r) * BLK:(NB * m + r + 1) * BLK,
                            h * DH:(h + 1) * DH] = ctxb[m * BLK:(m + 1) * BLK]

        HD = DM // 2
        rs_rdmas = []
        for off in (1, 2, 3):
            p = lax.rem(my + off, N_DEV)
            pc = jnp.dot(ctx_buf[pl.ds(p * QROWS, QROWS), :], wo_bf[:, :],
                         preferred_element_type=jnp.float32)
            p_send[pl.ds(p * QROWS, QROWS), :] = pc.astype(jnp.bfloat16)
            for hf in (0, 1):
                rd = pltpu.make_async_remote_copy(
                    src_ref=p_send.at[pl.ds(p * QROWS, QROWS),
                                      hf * HD:(hf + 1) * HD],
                    dst_ref=rs_recv.at[off - 1, :, hf * HD:(hf + 1) * HD],
                    send_sem=rs_send_sems.at[off - 1, hf],
                    recv_sem=rs_recv_sems.at[off - 1, hf],
                    device_id=(p,),
                    device_id_type=_MESH)
                rd.start()
                rs_rdmas.append((hf, rd))
        acc_own = jnp.dot(ctx_buf[pl.ds(my * QROWS, QROWS), :], wo_bf[:, :],
                          preferred_element_type=jnp.float32)

        ag_rdmas = []
        accs = []
        for hf in (0, 1):
            for h, rd in rs_rdmas:
                if h == hf:
                    rd.wait_recv()
            acc_h = acc_own[:, hf * HD:(hf + 1) * HD]
            for o in range(3):
                acc_h = acc_h + rs_recv[o, :, hf * HD:(hf + 1) * HD].astype(
                    jnp.float32)
            accs.append(acc_h)
            ag_own[:, hf * HD:(hf + 1) * HD] = acc_h.astype(jnp.bfloat16)
            for off in (1, 2, 3):
                p = lax.rem(my + off, N_DEV)
                rd = pltpu.make_async_remote_copy(
                    src_ref=ag_own.at[:, hf * HD:(hf + 1) * HD],
                    dst_ref=ag_recv.at[off - 1, :, hf * HD:(hf + 1) * HD],
                    send_sem=ag_send_sems.at[off - 1, hf],
                    recv_sem=ag_recv_sems.at[off - 1, hf],
                    device_id=(p,),
                    device_id_type=_MESH)
                rd.start()
                ag_rdmas.append(rd)
        out_ref[0, pl.ds(my * QROWS, QROWS), :] = jnp.concatenate(accs, axis=1)
        for rd in ag_rdmas:
            rd.wait_recv()
        for off in (1, 2, 3):
            j = lax.rem(my + (N_DEV - off), N_DEV)
            out_ref[0, pl.ds(j * QROWS, QROWS), :] = (
                ag_recv[off - 1].astype(jnp.float32))

        for rd in kv_rdmas + ag_rdmas:
            rd.wait_send()
        for _, rd in rs_rdmas:
            rd.wait_send()

    bf = jnp.bfloat16
    f32 = jnp.float32
    return pl.pallas_call(
        body,
        out_shape=jax.ShapeDtypeStruct((1, SQ, DM), f32),
        in_specs=[pl.BlockSpec(memory_space=_ANY)] * 5,
        out_specs=pl.BlockSpec(memory_space=pltpu.VMEM),
        scratch_shapes=[
            pltpu.VMEM((2, NB, H_TOT, GKV, DH), bf),
            pltpu.VMEM((2, NB, H_LOC, GKV, DH), jnp.float8_e4m3fn),
            pltpu.VMEM((NB, H_LOC, N_DEV * GKV, DH), bf),
            pltpu.VMEM((NB, H_LOC, N_DEV * GKV, DH), bf),
            pltpu.VMEM((2, NB, H_LOC, GKV, DH), jnp.float8_e4m3fn),
            pltpu.VMEM((2, H_TOT, BLK, DH), f32),
            pltpu.VMEM((1, 256, DM), f32),
            pltpu.VMEM((DM, DM), bf),
            pltpu.VMEM((DM, DM), bf),
            pltpu.VMEM((DM, DM), bf),
            pltpu.VMEM((SQ, H_LOC * DH), bf),
            pltpu.VMEM((SQ, H_LOC * DH), bf),
            pltpu.VMEM((SQ, DM), bf),
            pltpu.VMEM((3, QROWS, DM), bf),
            pltpu.VMEM((QROWS, DM), bf),
            pltpu.VMEM((3, QROWS, DM), bf),
            pltpu.SemaphoreType.DMA((2,)),
            pltpu.SemaphoreType.DMA((1,)),
            pltpu.SemaphoreType.DMA((2, NB)),
            pltpu.SemaphoreType.DMA((3, NB)),
            pltpu.SemaphoreType.DMA((3, NB)),
            pltpu.SemaphoreType.DMA((3, NB)),
            pltpu.SemaphoreType.DMA((3, NB)),
            pltpu.SemaphoreType.DMA((3, 2)),
            pltpu.SemaphoreType.DMA((3, 2)),
            pltpu.SemaphoreType.DMA((3, 2)),
            pltpu.SemaphoreType.DMA((3, 2)),
        ],
        compiler_params=_Params(collective_id=0,
                                vmem_limit_bytes=67000000),
    )(x, Wq, K_ext, V_ext, Wo)
